# baseline (speedup 1.0000x reference)
import functools

import jax
import jax.numpy as jnp
from jax.experimental import pallas as pl
from jax.experimental.pallas import tpu as pltpu

_B, _H, _N, _D = 1, 16, 2048, 64
_K = 32
_QBLK = 128
_NEG = -1e30


def _body(mask_ref, q_ref, k_ref, v_ref, o_ref):
    x = mask_ref[0]
    iota = jax.lax.broadcasted_iota(jnp.int32, (_QBLK, _N), 1)

    def step(_, x):
        m = jnp.max(x, axis=1, keepdims=True)
        fi = jnp.min(jnp.where(x >= m, iota, _N), axis=1, keepdims=True)
        return jnp.where(iota == fi, _NEG, x)

    x = jax.lax.fori_loop(0, _K, step, x, unroll=True)
    bias = jnp.where(x < 0.0, 0.0, _NEG)

    for h in range(_H):
        q = q_ref[0, h] * 0.125
        k = k_ref[0, h]
        v = v_ref[0, h]
        s = jax.lax.dot_general(
            q, k, (((1,), (1,)), ((), ())), preferred_element_type=jnp.float32
        )
        e = jnp.exp(s + bias)
        p = e / jnp.sum(e, axis=1, keepdims=True)
        o_ref[0, h] = jax.lax.dot_general(
            p, v, (((1,), (0,)), ((), ())), preferred_element_type=jnp.float32
        )


@jax.jit
def kernel(query, key, value, attention_mask):
    grid = (_N // _QBLK,)
    return pl.pallas_call(
        _body,
        grid=grid,
        in_specs=[
            pl.BlockSpec((1, _QBLK, _N), lambda i: (0, i, 0)),
            pl.BlockSpec((1, _H, _QBLK, _D), lambda i: (0, 0, i, 0)),
            pl.BlockSpec((1, _H, _N, _D), lambda i: (0, 0, 0, 0)),
            pl.BlockSpec((1, _H, _N, _D), lambda i: (0, 0, 0, 0)),
        ],
        out_specs=pl.BlockSpec((1, _H, _QBLK, _D), lambda i: (0, 0, i, 0)),
        out_shape=jax.ShapeDtypeStruct((_B, _H, _N, _D), jnp.float32),
        compiler_params=pltpu.CompilerParams(
            dimension_semantics=("arbitrary",),
        ),
    )(attention_mask, query, key, value)


# speedup vs baseline: 1.1472x; 1.1472x over previous
import functools

import jax
import jax.numpy as jnp
from jax.experimental import pallas as pl
from jax.experimental.pallas import tpu as pltpu

_B, _H, _N, _D = 1, 16, 2048, 64
_K = 32
_QBLK = 128
_NEG = -1e30


def _body(mask_ref, q_ref, k_ref, v_ref, o_ref):
    x = mask_ref[0]
    iota = jax.lax.broadcasted_iota(jnp.int32, (_QBLK, _N), 1)

    def step(_, x):
        m = jnp.max(x, axis=1, keepdims=True)
        fi = jnp.min(jnp.where(x >= m, iota, _N), axis=1, keepdims=True)
        return jnp.where(iota == fi, _NEG, x)

    x = jax.lax.fori_loop(0, _K, step, x, unroll=True)
    bias = jnp.where(x < 0.0, 0.0, _NEG)

    for h in range(_H):
        q = q_ref[0, h] * 0.125
        k = k_ref[0, h]
        v = v_ref[0, h]
        s = jax.lax.dot_general(
            q, k, (((1,), (1,)), ((), ())), preferred_element_type=jnp.float32
        )
        e = jnp.exp(s + bias)
        r = 1.0 / jnp.sum(e, axis=1, keepdims=True)
        o = jax.lax.dot_general(
            e, v, (((1,), (0,)), ((), ())), preferred_element_type=jnp.float32
        )
        o_ref[0, h] = o * r


@jax.jit
def kernel(query, key, value, attention_mask):
    grid = (_N // _QBLK,)
    return pl.pallas_call(
        _body,
        grid=grid,
        in_specs=[
            pl.BlockSpec((1, _QBLK, _N), lambda i: (0, i, 0)),
            pl.BlockSpec((1, _H, _QBLK, _D), lambda i: (0, 0, i, 0)),
            pl.BlockSpec((1, _H, _N, _D), lambda i: (0, 0, 0, 0)),
            pl.BlockSpec((1, _H, _N, _D), lambda i: (0, 0, 0, 0)),
        ],
        out_specs=pl.BlockSpec((1, _H, _QBLK, _D), lambda i: (0, 0, i, 0)),
        out_shape=jax.ShapeDtypeStruct((_B, _H, _N, _D), jnp.float32),
        compiler_params=pltpu.CompilerParams(
            dimension_semantics=("arbitrary",),
        ),
    )(attention_mask, query, key, value)
